# baseline (speedup 1.0000x reference)
import jax
import jax.numpy as jnp
from jax import lax
from jax.experimental import pallas as pl
from jax.experimental.pallas import tpu as pltpu
from jax.experimental.pallas import tpu_sc as plsc

VOCAB = 100000
DIM = 64
BATCH = 4096
HIST = 50
FEAT = 16

NC = 2
NS = 16
NW = NC * NS
BPW = BATCH // NW
GS = 2
NG = BPW // GS
NBUF = 8
NT = NG // NBUF

_PK_IN = 4096
_PK_OUT = _PK_IN // 8
_PK_STEPS = 25
_PK_ROWS = _PK_STEPS * _PK_OUT
_MLP_BLK = 512


def _project_pack(table, W1):
    def body(t_ref, w_ref, o_ref):
        g = jax.lax.dot_general(w_ref[...], t_ref[...],
                                (((0,), (0,)), ((), ())),
                                preferred_element_type=jnp.float32)
        parts = [g[:, k * _PK_OUT:(k + 1) * _PK_OUT].T for k in range(8)]
        o_ref[...] = jnp.concatenate(parts, axis=1)

    tab_t = table.T
    return pl.pallas_call(
        body,
        grid=(_PK_STEPS,),
        in_specs=[
            pl.BlockSpec((DIM, _PK_IN), lambda i: (0, i)),
            pl.BlockSpec((DIM, FEAT), lambda i: (0, 0)),
        ],
        out_specs=pl.BlockSpec((_PK_OUT, 8 * FEAT), lambda i: (i, 0)),
        out_shape=jax.ShapeDtypeStruct((_PK_ROWS, 8 * FEAT), jnp.float32),
    )(tab_t, W1)


def _sc_body(g_hbm, x_hbm, b1_hbm, out_hbm, idx_v, rows_v, out_v, b1_v,
             *sems):
    wid = lax.axis_index("s") * NC + lax.axis_index("c")

    pltpu.sync_copy(x_hbm.at[pl.ds(wid * NG, NG)], idx_v)
    pltpu.sync_copy(b1_hbm, b1_v)
    b1 = b1_v[...]
    inv_h = jnp.float32(1.0 / HIST)

    for b in range(NBUF):
        pltpu.async_copy(g_hbm.at[idx_v.at[b]], rows_v.at[b], sems[b])

    def outer(t, carry):
        for b in range(NBUF):
            g = t * NBUF + b
            pltpu.make_async_copy(g_hbm.at[idx_v.at[g]], rows_v.at[b],
                                  sems[b]).wait()
            for e in range(GS):
                base_r = e * HIST
                a0 = rows_v[b, base_r, :]
                a1 = rows_v[b, base_r + 1, :]
                for j in range(2, HIST, 2):
                    a0 = a0 + rows_v[b, base_r + j, :]
                    a1 = a1 + rows_v[b, base_r + j + 1, :]
                out_v[g * GS + e] = jnp.maximum((a0 + a1) * inv_h + b1, 0.0)
            @pl.when(g + NBUF < NG)
            def _():
                pltpu.async_copy(g_hbm.at[idx_v.at[g + NBUF]], rows_v.at[b],
                                 sems[b])
        return carry

    lax.fori_loop(0, NT, outer, 0)
    pltpu.sync_copy(out_v, out_hbm.at[pl.ds(wid * BPW, BPW)])


def _sc_pool_hidden(g_lin, x2, b1):
    mesh = plsc.VectorSubcoreMesh(core_axis_name="c", subcore_axis_name="s")
    kfn = pl.kernel(
        _sc_body,
        out_type=jax.ShapeDtypeStruct((BATCH, FEAT), jnp.float32),
        mesh=mesh,
        scratch_types=[
            pltpu.VMEM((NG, GS * HIST), jnp.int32),
            pltpu.VMEM((NBUF, GS * HIST, FEAT), jnp.float32),
            pltpu.VMEM((BPW, FEAT), jnp.float32),
            pltpu.VMEM((FEAT,), jnp.float32),
        ] + [pltpu.SemaphoreType.DMA] * NBUF,
        compiler_params=pltpu.CompilerParams(use_tc_tiling_on_sc=False),
    )
    return kfn(g_lin, x2, b1)


def _fc2(h, W2, b2):
    def body(h_ref, w_ref, b_ref, o_ref):
        o_ref[...] = jnp.dot(h_ref[...], w_ref[...],
                             preferred_element_type=jnp.float32) + b_ref[...]

    return pl.pallas_call(
        body,
        grid=(BATCH // _MLP_BLK,),
        in_specs=[
            pl.BlockSpec((_MLP_BLK, FEAT), lambda i: (i, 0)),
            pl.BlockSpec((FEAT, 1), lambda i: (0, 0)),
            pl.BlockSpec((1, 1), lambda i: (0, 0)),
        ],
        out_specs=pl.BlockSpec((_MLP_BLK, 1), lambda i: (i, 0)),
        out_shape=jax.ShapeDtypeStruct((BATCH, 1), jnp.float32),
    )(h, W2, b2.reshape(1, 1))


def kernel(x, table, W1, b1, W2, b2):
    xi = x.astype(jnp.int32)
    xr = 8 * (_PK_OUT * (xi // _PK_IN) + xi % _PK_OUT) + (xi // _PK_OUT) % 8
    x2 = xr.reshape(BATCH // GS, GS * HIST)
    g_lin = _project_pack(table, W1).reshape(_PK_ROWS * 8, FEAT)
    h = _sc_pool_hidden(g_lin, x2, b1)
    return _fc2(h, W2, b2)


# speedup vs baseline: 1.0430x; 1.0430x over previous
import jax
import jax.numpy as jnp
from jax import lax
from jax.experimental import pallas as pl
from jax.experimental.pallas import tpu as pltpu
from jax.experimental.pallas import tpu_sc as plsc

VOCAB = 100000
DIM = 64
BATCH = 4096
HIST = 50
FEAT = 16

NC = 2
NS = 16
NW = NC * NS
BPW = BATCH // NW
GS = 2
NG = BPW // GS
NBUF = 8
NT = NG // NBUF

_PK_IN = 8192
_PK_OUT = _PK_IN // 8
_PK_STEPS = 13
_PK_ROWS = _PK_STEPS * _PK_OUT
_MLP_BLK = 512


def _project_pack(table, W1):
    def body(t_ref, w_ref, o_ref):
        g = jax.lax.dot_general(w_ref[...], t_ref[...],
                                (((0,), (0,)), ((), ())),
                                preferred_element_type=jnp.float32)
        parts = [g[:, k * _PK_OUT:(k + 1) * _PK_OUT].T for k in range(8)]
        o_ref[...] = jnp.concatenate(parts, axis=1)

    tab_t = table.T
    return pl.pallas_call(
        body,
        grid=(_PK_STEPS,),
        in_specs=[
            pl.BlockSpec((DIM, _PK_IN), lambda i: (0, i)),
            pl.BlockSpec((DIM, FEAT), lambda i: (0, 0)),
        ],
        out_specs=pl.BlockSpec((_PK_OUT, 8 * FEAT), lambda i: (i, 0)),
        out_shape=jax.ShapeDtypeStruct((_PK_ROWS, 8 * FEAT), jnp.float32),
    )(tab_t, W1)


_GDN = lax.GatherDimensionNumbers(offset_dims=(), collapsed_slice_dims=(0,),
                                  start_index_map=(0,))


def _lane_shuffle(v, lane, mask):
    return lax.gather(v, (lane ^ mask)[:, None], _GDN, (1,),
                      mode=lax.GatherScatterMode.PROMISE_IN_BOUNDS)


def _sc_body(g_hbm, x_hbm, b1_hbm, w2_hbm, b2_hbm, out_hbm,
             idx_v, rows_v, out_v, b1_v, w2_v, b2_v, *sems):
    wid = lax.axis_index("s") * NC + lax.axis_index("c")

    pltpu.sync_copy(x_hbm.at[pl.ds(wid * NG, NG)], idx_v)
    pltpu.sync_copy(b1_hbm, b1_v)
    pltpu.sync_copy(w2_hbm, w2_v)
    pltpu.sync_copy(b2_hbm, b2_v)
    b1 = b1_v[...]
    w2 = w2_v[...]
    b2 = b2_v[...]
    lane = lax.iota(jnp.int32, 16)
    inv_h = jnp.float32(1.0 / HIST)

    for b in range(NBUF):
        pltpu.async_copy(g_hbm.at[idx_v.at[b]], rows_v.at[b], sems[b])

    def outer(t, carry):
        res = jnp.zeros((16,), jnp.float32)
        for b in range(NBUF):
            g = t * NBUF + b
            pltpu.make_async_copy(g_hbm.at[idx_v.at[g]], rows_v.at[b],
                                  sems[b]).wait()
            for e in range(GS):
                base_r = e * HIST
                a0 = rows_v[b, base_r, :]
                a1 = rows_v[b, base_r + 1, :]
                for j in range(2, HIST, 2):
                    a0 = a0 + rows_v[b, base_r + j, :]
                    a1 = a1 + rows_v[b, base_r + j + 1, :]
                r = jnp.maximum((a0 + a1) * inv_h + b1, 0.0) * w2
                for m in (8, 4, 2, 1):
                    r = r + _lane_shuffle(r, lane, m)
                res = jnp.where(lane == b * GS + e, r + b2, res)
            @pl.when(g + NBUF < NG)
            def _():
                pltpu.async_copy(g_hbm.at[idx_v.at[g + NBUF]], rows_v.at[b],
                                 sems[b])
        out_v[t] = res
        return carry

    lax.fori_loop(0, NT, outer, 0)
    pltpu.sync_copy(out_v, out_hbm.at[pl.ds(wid * NT, NT)])


def _sc_pool_mlp(g_lin, x2, b1, w2, b2b):
    mesh = plsc.VectorSubcoreMesh(core_axis_name="c", subcore_axis_name="s")
    kfn = pl.kernel(
        _sc_body,
        out_type=jax.ShapeDtypeStruct((NW * NT, 16), jnp.float32),
        mesh=mesh,
        scratch_types=[
            pltpu.VMEM((NG, GS * HIST), jnp.int32),
            pltpu.VMEM((NBUF, GS * HIST, FEAT), jnp.float32),
            pltpu.VMEM((NT, 16), jnp.float32),
            pltpu.VMEM((FEAT,), jnp.float32),
            pltpu.VMEM((FEAT,), jnp.float32),
            pltpu.VMEM((16,), jnp.float32),
        ] + [pltpu.SemaphoreType.DMA] * NBUF,
        compiler_params=pltpu.CompilerParams(use_tc_tiling_on_sc=False),
    )
    return kfn(g_lin, x2, b1, w2, b2b)


def kernel(x, table, W1, b1, W2, b2):
    xi = x.astype(jnp.int32)
    xr = 8 * (_PK_OUT * (xi // _PK_IN) + xi % _PK_OUT) + (xi // _PK_OUT) % 8
    x2 = xr.reshape(BATCH // GS, GS * HIST)
    g_lin = _project_pack(table, W1).reshape(_PK_ROWS * 8, FEAT)
    out = _sc_pool_mlp(g_lin, x2, b1, W2.reshape(FEAT),
                       jnp.broadcast_to(b2, (16,)))
    return out.reshape(BATCH, 1)


# speedup vs baseline: 1.1801x; 1.1314x over previous
import jax
import jax.numpy as jnp
from jax import lax
from jax.experimental import pallas as pl
from jax.experimental.pallas import tpu as pltpu
from jax.experimental.pallas import tpu_sc as plsc

VOCAB = 100000
DIM = 64
BATCH = 4096
HIST = 50
FEAT = 16

NC = 2
NS = 16
NW = NC * NS
BPW = BATCH // NW
GS = 2
NG = BPW // GS
NBUF = 8
NT = NG // NBUF

_PK_IN = 8192
_PK_OUT = _PK_IN // 8
_PK_STEPS = 13
_PK_ROWS = _PK_STEPS * _PK_OUT
_MLP_BLK = 512


def _project_pack(table, W1):
    def body(t_ref, w_ref, o_ref):
        g = jax.lax.dot_general(w_ref[...], t_ref[...],
                                (((0,), (0,)), ((), ())),
                                preferred_element_type=jnp.float32)
        parts = [g[:, k * _PK_OUT:(k + 1) * _PK_OUT].T for k in range(8)]
        o_ref[...] = jnp.concatenate(parts, axis=1)

    tab_t = table.T
    return pl.pallas_call(
        body,
        grid=(_PK_STEPS,),
        in_specs=[
            pl.BlockSpec((DIM, _PK_IN), lambda i: (0, i)),
            pl.BlockSpec((DIM, FEAT), lambda i: (0, 0)),
        ],
        out_specs=pl.BlockSpec((_PK_OUT, 8 * FEAT), lambda i: (i, 0)),
        out_shape=jax.ShapeDtypeStruct((_PK_ROWS, 8 * FEAT), jnp.float32),
    )(tab_t, W1)


def _sc_body(g_hbm, x_hbm, b1_hbm, w2_hbm, b2_hbm, out_hbm,
             idx_v, rows_v, out_v, b1_v, w2_v, b2_v, tr_v, *sems):
    wid = lax.axis_index("s") * NC + lax.axis_index("c")

    pltpu.sync_copy(x_hbm.at[pl.ds(wid * NG, NG)], idx_v)
    pltpu.sync_copy(b1_hbm, b1_v)
    pltpu.sync_copy(w2_hbm, w2_v)
    pltpu.sync_copy(b2_hbm, b2_v)
    b1 = b1_v[...]
    w2 = w2_v[...]
    b2 = b2_v[...]
    lane = lax.iota(jnp.int32, 16)
    inv_h = jnp.float32(1.0 / HIST)

    for b in range(NBUF):
        pltpu.async_copy(g_hbm.at[idx_v.at[b]], rows_v.at[b], sems[b])

    def outer(t, carry):
        for b in range(NBUF):
            g = t * NBUF + b
            pltpu.make_async_copy(g_hbm.at[idx_v.at[g]], rows_v.at[b],
                                  sems[b]).wait()
            for e in range(GS):
                base_r = e * HIST
                a0 = rows_v[b, base_r, :]
                a1 = rows_v[b, base_r + 1, :]
                for j in range(2, HIST, 2):
                    a0 = a0 + rows_v[b, base_r + j, :]
                    a1 = a1 + rows_v[b, base_r + j + 1, :]
                r = jnp.maximum((a0 + a1) * inv_h + b1, 0.0) * w2
                plsc.store_scatter(
                    tr_v, [lane, jnp.full((16,), b * GS + e, jnp.int32)], r)
            @pl.when(g + NBUF < NG)
            def _():
                pltpu.async_copy(g_hbm.at[idx_v.at[g + NBUF]], rows_v.at[b],
                                 sems[b])
        s0 = tr_v[0, :] + tr_v[1, :]
        s1 = tr_v[2, :] + tr_v[3, :]
        for j in range(4, FEAT, 4):
            s0 = s0 + (tr_v[j, :] + tr_v[j + 1, :])
            s1 = s1 + (tr_v[j + 2, :] + tr_v[j + 3, :])
        out_v[t] = s0 + s1 + b2
        return carry

    lax.fori_loop(0, NT, outer, 0)
    pltpu.sync_copy(out_v, out_hbm.at[pl.ds(wid * NT, NT)])


def _sc_pool_mlp(g_lin, x2, b1, w2, b2b):
    mesh = plsc.VectorSubcoreMesh(core_axis_name="c", subcore_axis_name="s")
    kfn = pl.kernel(
        _sc_body,
        out_type=jax.ShapeDtypeStruct((NW * NT, 16), jnp.float32),
        mesh=mesh,
        scratch_types=[
            pltpu.VMEM((NG, GS * HIST), jnp.int32),
            pltpu.VMEM((NBUF, GS * HIST, FEAT), jnp.float32),
            pltpu.VMEM((NT, 16), jnp.float32),
            pltpu.VMEM((FEAT,), jnp.float32),
            pltpu.VMEM((FEAT,), jnp.float32),
            pltpu.VMEM((16,), jnp.float32),
            pltpu.VMEM((FEAT, 16), jnp.float32),
        ] + [pltpu.SemaphoreType.DMA] * NBUF,
        compiler_params=pltpu.CompilerParams(use_tc_tiling_on_sc=False,
                                             needs_layout_passes=False),
    )
    return kfn(g_lin, x2, b1, w2, b2b)


def kernel(x, table, W1, b1, W2, b2):
    xi = x.astype(jnp.int32)
    xr = 8 * (_PK_OUT * (xi // _PK_IN) + xi % _PK_OUT) + (xi // _PK_OUT) % 8
    x2 = xr.reshape(BATCH // GS, GS * HIST)
    g_lin = _project_pack(table, W1).reshape(_PK_ROWS * 8, FEAT)
    out = _sc_pool_mlp(g_lin, x2, b1, W2.reshape(FEAT),
                       jnp.broadcast_to(b2, (16,)))
    return out.reshape(BATCH, 1)
